# Initial kernel scaffold; baseline (speedup 1.0000x reference)
import jax
import jax.numpy as jnp
from jax.experimental import pallas as pl


def kernel(node_sca, node_vec, edge_sca, edge_vec, edge_index, node_pos, sca_attn_W, sca_attn_b, vec_attn_W, vec_attn_b, node_W, node_b, edge_W, edge_b, node_sca_W, node_sca_b, edge_sca_W, edge_sca_b, node_vec_W, edge_vec_W, lv_W, lv2_W, ls_W, gate_W, gate_b, dir_W):
    raise NotImplementedError("write your pallas kernel here")


# speedup vs baseline: 3.0015x; 3.0015x over previous
import functools

import jax
import jax.numpy as jnp
from jax.experimental import pallas as pl

N_NODES = 100000
N_EDGES = 1600000
OUT = 16
EPS = 1e-6

BN = 1000
BE = 2000


def _node_proj_kernel(ns_ref, nv_ref,
                      w1_ref, w2_ref, nw_ref, nb_ref, nsw_ref, nsb_ref,
                      nvw_ref, vaw_ref, vab_ref,
                      p1_ref, p2_ref, nf_ref, nsh_ref,
                      nvh0_ref, nvh1_ref, nvh2_ref,
                      av0_ref, av1_ref, av2_ref):
    ns = ns_ref[...]
    nv = nv_ref[...]
    p1_ref[...] = jnp.dot(ns, w1_ref[...], preferred_element_type=jnp.float32)
    p2_ref[...] = jnp.dot(ns, w2_ref[...], preferred_element_type=jnp.float32)
    nf_ref[...] = jnp.dot(ns, nw_ref[...], preferred_element_type=jnp.float32) + nb_ref[...]
    nsh_ref[...] = jnp.dot(ns, nsw_ref[...], preferred_element_type=jnp.float32) + nsb_ref[...]
    nvw = nvw_ref[...]
    vaw = vaw_ref[...]
    vab = vab_ref[...]
    for d, (nvh_ref, av_ref) in enumerate(((nvh0_ref, av0_ref),
                                           (nvh1_ref, av1_ref),
                                           (nvh2_ref, av2_ref))):
        c0 = nv[:, d:d + 1]
        c1 = nv[:, 3 + d:4 + d]
        nvh_ref[...] = c0 * nvw[0:1, :] + c1 * nvw[1:2, :]
        av_ref[...] = c0 * vaw[0:1, :] + c1 * vaw[1:2, :] + vab


def _logits_kernel(p1s_ref, p2d_ref, ps_ref, pd_ref, w3_ref, b_ref, a_ref):
    diff = ps_ref[...] - pd_ref[...]
    dist = jnp.sqrt(jnp.sum(diff * diff, axis=1, keepdims=True) + 1e-12)
    a_ref[...] = p1s_ref[...] + p2d_ref[...] + dist * w3_ref[...] + b_ref[...]


def _edge_kernel(a_ref, ms_ref, nfs_ref, es_ref, nshs_ref, ev_ref,
                 nvh0_ref, nvh1_ref, nvh2_ref,
                 avs0_ref, avs1_ref, avs2_ref,
                 avd0_ref, avd1_ref, avd2_ref,
                 ew_ref, eb_ref, esw_ref, esb_ref, evw_ref,
                 ex_ref, fex_ref, m0_ref, m1_ref, m2_ref):
    ex = jnp.exp(a_ref[...] - ms_ref[...])
    ex_ref[...] = ex
    es = es_ref[...]
    ef = jnp.dot(es, ew_ref[...], preferred_element_type=jnp.float32) + eb_ref[...]
    fex_ref[...] = nfs_ref[...] * ef * ex
    esh = jnp.dot(es, esw_ref[...], preferred_element_type=jnp.float32) + esb_ref[...]
    dot = (jnp.sum(avs0_ref[...] * avd0_ref[...], axis=1, keepdims=True)
           + jnp.sum(avs1_ref[...] * avd1_ref[...], axis=1, keepdims=True)
           + jnp.sum(avs2_ref[...] * avd2_ref[...], axis=1, keepdims=True))
    alpha_vec = jax.nn.sigmoid(dot)
    nsh = nshs_ref[...]
    ev = ev_ref[...]
    evw = evw_ref[...]
    for d, (nvh_ref, m_ref) in enumerate(((nvh0_ref, m0_ref),
                                          (nvh1_ref, m1_ref),
                                          (nvh2_ref, m2_ref))):
        evh = ev[:, d:d + 1] * evw
        m_ref[...] = (nsh * evh + nvh_ref[...] * esh) * alpha_vec


def _mapper_kernel(esca_ref, ev0_ref, ev1_ref, ev2_ref,
                   lv_ref, lv2_ref, ls1_ref, ls2_ref, gw_ref, gb_ref, dw_ref,
                   osca_ref, ov0_ref, ov1_ref, ov2_ref):
    lv = lv_ref[...]
    vi0 = jnp.dot(ev0_ref[...], lv, preferred_element_type=jnp.float32)
    vi1 = jnp.dot(ev1_ref[...], lv, preferred_element_type=jnp.float32)
    vi2 = jnp.dot(ev2_ref[...], lv, preferred_element_type=jnp.float32)
    vec_norm = jnp.sqrt(vi0 * vi0 + vi1 * vi1 + vi2 * vi2 + 1e-12)
    out_sca = (jnp.dot(vec_norm, ls1_ref[...], preferred_element_type=jnp.float32)
               + jnp.dot(esca_ref[...], ls2_ref[...], preferred_element_type=jnp.float32))
    lv2 = lv2_ref[...]
    ov0 = jnp.dot(vi0, lv2, preferred_element_type=jnp.float32)
    ov1 = jnp.dot(vi1, lv2, preferred_element_type=jnp.float32)
    ov2 = jnp.dot(vi2, lv2, preferred_element_type=jnp.float32)
    gates = jax.nn.sigmoid(jnp.dot(out_sca, gw_ref[...],
                                   preferred_element_type=jnp.float32) + gb_ref[...])
    ov0 = gates * ov0
    ov1 = gates * ov1
    ov2 = gates * ov2
    dwm = dw_ref[...]
    d0 = jnp.dot(ov0, dwm, preferred_element_type=jnp.float32)
    d1 = jnp.dot(ov1, dwm, preferred_element_type=jnp.float32)
    d2 = jnp.dot(ov2, dwm, preferred_element_type=jnp.float32)
    dot = ov0 * d0 + ov1 * d1 + ov2 * d2
    mask = (dot >= 0).astype(jnp.float32)
    dns = d0 * d0 + d1 * d1 + d2 * d2
    scale = (1.0 - mask) * (dot / (dns + EPS))
    ov0_ref[...] = 0.2 * ov0 + 0.8 * (ov0 - scale * d0)
    ov1_ref[...] = 0.2 * ov1 + 0.8 * (ov1 - scale * d1)
    ov2_ref[...] = 0.2 * ov2 + 0.8 * (ov2 - scale * d2)
    osca_ref[...] = jnp.where(out_sca >= 0, out_sca, 0.01 * out_sca)


def _nblock(c):
    return pl.BlockSpec((BN, c), lambda i: (i, 0))


def _eblock(c):
    return pl.BlockSpec((BE, c), lambda i: (i, 0))


def _wblock(r, c):
    return pl.BlockSpec((r, c), lambda i: (0, 0))


@jax.jit
def kernel(node_sca, node_vec, edge_sca, edge_vec, edge_index, node_pos,
           sca_attn_W, sca_attn_b, vec_attn_W, vec_attn_b,
           node_W, node_b, edge_W, edge_b,
           node_sca_W, node_sca_b, edge_sca_W, edge_sca_b,
           node_vec_W, edge_vec_W,
           lv_W, lv2_W, ls_W, gate_W, gate_b, dir_W):
    f32 = jnp.float32
    src = edge_index[0]
    dst = edge_index[1]
    nv_flat = node_vec.reshape(N_NODES, 6)
    ev_flat = edge_vec.reshape(N_EDGES, 3)

    n16 = jax.ShapeDtypeStruct((N_NODES, OUT), f32)
    node_outs = pl.pallas_call(
        _node_proj_kernel,
        grid=(N_NODES // BN,),
        in_specs=[_nblock(13), _nblock(6),
                  _wblock(13, OUT), _wblock(13, OUT), _wblock(13, OUT),
                  _wblock(1, OUT), _wblock(13, OUT), _wblock(1, OUT),
                  _wblock(2, OUT), _wblock(2, OUT), _wblock(1, OUT)],
        out_specs=[_nblock(OUT)] * 10,
        out_shape=[n16] * 10,
    )(node_sca, nv_flat,
      sca_attn_W[:, :13].T, sca_attn_W[:, 13:26].T, node_W.T,
      node_b.reshape(1, OUT), node_sca_W.T, node_sca_b.reshape(1, OUT),
      node_vec_W.T, vec_attn_W.T, vec_attn_b.reshape(1, OUT))
    p1, p2, nf, nsh, nvh0, nvh1, nvh2, av0, av1, av2 = node_outs

    take = functools.partial(jnp.take, axis=0)

    e16 = jax.ShapeDtypeStruct((N_EDGES, OUT), f32)
    a = pl.pallas_call(
        _logits_kernel,
        grid=(N_EDGES // BE,),
        in_specs=[_eblock(OUT), _eblock(OUT), _eblock(3), _eblock(3),
                  _wblock(1, OUT), _wblock(1, OUT)],
        out_specs=_eblock(OUT),
        out_shape=e16,
    )(take(p1, src), take(p2, dst), take(node_pos, src), take(node_pos, dst),
      sca_attn_W[:, 26].reshape(1, OUT), sca_attn_b.reshape(1, OUT))

    m = jax.ops.segment_max(a, src, num_segments=N_NODES)
    m = jnp.where(jnp.isfinite(m), m, 0.0)

    ex, fex, m0, m1, m2 = pl.pallas_call(
        _edge_kernel,
        grid=(N_EDGES // BE,),
        in_specs=[_eblock(OUT), _eblock(OUT), _eblock(OUT), _eblock(4),
                  _eblock(OUT), _eblock(3),
                  _eblock(OUT), _eblock(OUT), _eblock(OUT),
                  _eblock(OUT), _eblock(OUT), _eblock(OUT),
                  _eblock(OUT), _eblock(OUT), _eblock(OUT),
                  _wblock(4, OUT), _wblock(1, OUT), _wblock(4, OUT),
                  _wblock(1, OUT), _wblock(1, OUT)],
        out_specs=[_eblock(OUT)] * 5,
        out_shape=[e16] * 5,
    )(a, take(m, src), take(nf, src), edge_sca, take(nsh, src), ev_flat,
      take(nvh0, src), take(nvh1, src), take(nvh2, src),
      take(av0, src), take(av1, src), take(av2, src),
      take(av0, dst), take(av1, dst), take(av2, dst),
      edge_W.T, edge_b.reshape(1, OUT), edge_sca_W.T,
      edge_sca_b.reshape(1, OUT), edge_vec_W[:, 0].reshape(1, OUT))

    seg = functools.partial(jax.ops.segment_sum, segment_ids=src,
                            num_segments=N_NODES)
    denom = seg(ex)
    emb_sca = seg(fex) / (denom + 1e-16)
    ev0 = seg(m0)
    ev1 = seg(m1)
    ev2 = seg(m2)

    out_sca, ov0, ov1, ov2 = pl.pallas_call(
        _mapper_kernel,
        grid=(N_NODES // BN,),
        in_specs=[_nblock(OUT)] * 4 + [_wblock(OUT, OUT), _wblock(OUT, OUT),
                                       _wblock(OUT, OUT), _wblock(OUT, OUT),
                                       _wblock(OUT, OUT), _wblock(1, OUT),
                                       _wblock(OUT, OUT)],
        out_specs=[_nblock(OUT)] * 4,
        out_shape=[n16] * 4,
    )(emb_sca, ev0, ev1, ev2,
      lv_W.T, lv2_W.T, ls_W[:, :OUT].T, ls_W[:, OUT:].T,
      gate_W.T, gate_b.reshape(1, OUT), dir_W.T)

    out_vec = jnp.stack([ov0, ov1, ov2], axis=-1)
    return out_sca, out_vec
